# baseline (speedup 1.0000x reference)
import functools

import jax
import jax.numpy as jnp
from jax import lax
from jax.experimental import pallas as pl
from jax.experimental.pallas import tpu as pltpu
from jax.experimental.pallas import tpu_sc as plsc

B = 16384
EMB = 32
NSIDE = 3
NMATCH = 20
NC, NS = 2, 16
NW = NC * NS
BW = B // NW
C = 128
NCHUNK = BW // C
ID_V = 1000000
WROWS = ID_V * NSIDE // EMB


def _sc_gather(qid, qcat, qbrand, mat, id_table, cat_table, brand_table,
               out_table, weight_table):
  mesh = plsc.VectorSubcoreMesh(core_axis_name="c", subcore_axis_name="s")

  @functools.partial(
      pl.kernel,
      mesh=mesh,
      compiler_params=pltpu.CompilerParams(use_tc_tiling_on_sc=False),
      out_type=[
          jax.ShapeDtypeStruct((B, EMB), jnp.float32),
          jax.ShapeDtypeStruct((B, EMB), jnp.float32),
          jax.ShapeDtypeStruct((B, EMB), jnp.float32),
          jax.ShapeDtypeStruct((B, EMB), jnp.float32),
          jax.ShapeDtypeStruct((B, EMB), jnp.float32),
          jax.ShapeDtypeStruct((B * NMATCH, EMB), jnp.float32),
      ],
      scratch_types=[
          pltpu.VMEM((C,), jnp.int32),
          pltpu.VMEM((C,), jnp.int32),
          pltpu.VMEM((C,), jnp.int32),
          pltpu.VMEM((C,), jnp.int32),
          pltpu.VMEM((C,), jnp.int32),
          pltpu.VMEM((NMATCH * C,), jnp.int32),
          pltpu.VMEM((C, EMB), jnp.float32),
          pltpu.VMEM((C, EMB), jnp.float32),
          pltpu.VMEM((C, EMB), jnp.float32),
          pltpu.VMEM((C, EMB), jnp.float32),
          pltpu.VMEM((C, EMB), jnp.float32),
          pltpu.VMEM((NMATCH * C, EMB), jnp.float32),
          pltpu.SemaphoreType.DMA,
      ],
  )
  def gather_kernel(qid_h, qcat_h, qbrand_h, mat_h, idt, catt, brandt, outt,
                    wtv, o_id, o_cat, o_brand, o_wa, o_wb, o_mat,
                    qid_v, qcat_v, qbrand_v, wia_v, wib_v, mat_v, id_v,
                    cat_v, brand_v, wa_v, wb_v, mrows_v, sem):
    wid = lax.axis_index("s") * NC + lax.axis_index("c")

    def chunk(c, carry):
      base = wid * BW + c * C
      pltpu.sync_copy(qid_h.at[pl.ds(base, C)], qid_v)
      pltpu.sync_copy(qcat_h.at[pl.ds(base, C)], qcat_v)
      pltpu.sync_copy(qbrand_h.at[pl.ds(base, C)], qbrand_v)
      pltpu.sync_copy(mat_h.at[pl.ds(base * NMATCH, NMATCH * C)], mat_v)
      for i in range(C // 16):
        t = qid_v[pl.ds(i * 16, 16)] * 3
        ra = jnp.minimum(lax.shift_right_logical(t, 5), WROWS - 2)
        wia_v[pl.ds(i * 16, 16)] = ra
        wib_v[pl.ds(i * 16, 16)] = ra + 1
      copies = [
          pltpu.async_copy(idt.at[qid_v], id_v, sem),
          pltpu.async_copy(catt.at[qcat_v], cat_v, sem),
          pltpu.async_copy(brandt.at[qbrand_v], brand_v, sem),
          pltpu.async_copy(wtv.at[wia_v], wa_v, sem),
          pltpu.async_copy(wtv.at[wib_v], wb_v, sem),
      ]
      for j in range(NMATCH):
        copies.append(pltpu.async_copy(outt.at[mat_v.at[pl.ds(j * C, C)]],
                                       mrows_v.at[pl.ds(j * C, C)], sem))
      for cp in copies:
        cp.wait()
      pltpu.sync_copy(id_v, o_id.at[pl.ds(base, C)])
      pltpu.sync_copy(cat_v, o_cat.at[pl.ds(base, C)])
      pltpu.sync_copy(brand_v, o_brand.at[pl.ds(base, C)])
      pltpu.sync_copy(wa_v, o_wa.at[pl.ds(base, C)])
      pltpu.sync_copy(wb_v, o_wb.at[pl.ds(base, C)])
      pltpu.sync_copy(mrows_v, o_mat.at[pl.ds(base * NMATCH, NMATCH * C)])
      return carry

    lax.fori_loop(0, NCHUNK, chunk, 0)

  return gather_kernel(qid, qcat, qbrand, mat, id_table, cat_table,
                       brand_table, out_table, weight_table)


R = 256


def _tc_body(id_ref, cat_ref, brand_ref, qid_ref, wa_ref, wb_ref, me_ref,
             out_ref):
  t = qid_ref[...] * 3
  ra = jnp.minimum(lax.shift_right_logical(t, 5), WROWS - 2)
  off = t - EMB * ra
  wwin = jnp.concatenate([wa_ref[...], wb_ref[...]], axis=1)
  lane = lax.broadcasted_iota(jnp.int32, (1, 2 * EMB), 1)
  zero = jnp.zeros_like(wwin)
  w0 = jnp.sum(jnp.where(lane == off, wwin, zero), axis=1, keepdims=True)
  w1 = jnp.sum(jnp.where(lane == off + 1, wwin, zero), axis=1, keepdims=True)
  w2 = jnp.sum(jnp.where(lane == off + 2, wwin, zero), axis=1, keepdims=True)
  wm = jnp.maximum(jnp.maximum(w0, w1), w2)
  e0, e1, e2 = jnp.exp(w0 - wm), jnp.exp(w1 - wm), jnp.exp(w2 - wm)
  inv = 1.0 / (e0 + e1 + e2)
  hidden = (e0 * id_ref[...] + e1 * cat_ref[...]
            + e2 * brand_ref[...]) * inv
  col = lax.broadcasted_iota(jnp.int32, (EMB, NMATCH * EMB), 1)
  row = lax.broadcasted_iota(jnp.int32, (EMB, NMATCH * EMB), 0)
  t_mat = (col % EMB == row).astype(jnp.float32)
  srow = lax.broadcasted_iota(jnp.int32, (NMATCH * EMB, NMATCH), 0)
  scol = lax.broadcasted_iota(jnp.int32, (NMATCH * EMB, NMATCH), 1)
  s_mat = (srow // EMB == scol).astype(jnp.float32)
  ht = jnp.dot(hidden, t_mat, preferred_element_type=jnp.float32)
  prod = me_ref[...] * ht
  out_ref[...] = jnp.dot(prod, s_mat, preferred_element_type=jnp.float32)


def _tc_compute(id_rows, cat_rows, brand_rows, qid2d, wa, wb, match_rows):
  return pl.pallas_call(
      _tc_body,
      grid=(B // R,),
      in_specs=[
          pl.BlockSpec((R, EMB), lambda i: (i, 0)),
          pl.BlockSpec((R, EMB), lambda i: (i, 0)),
          pl.BlockSpec((R, EMB), lambda i: (i, 0)),
          pl.BlockSpec((R, 1), lambda i: (i, 0)),
          pl.BlockSpec((R, EMB), lambda i: (i, 0)),
          pl.BlockSpec((R, EMB), lambda i: (i, 0)),
          pl.BlockSpec((R, NMATCH * EMB), lambda i: (i, 0)),
      ],
      out_specs=pl.BlockSpec((R, NMATCH), lambda i: (i, 0)),
      out_shape=jax.ShapeDtypeStruct((B, NMATCH), jnp.float32),
  )(id_rows, cat_rows, brand_rows, qid2d, wa, wb, match_rows)


def kernel(query_item_id, query_cat_id, query_brand_id, match, id_table,
           cat_table, brand_table, out_table, weight_table):
  qid = query_item_id.reshape(-1).astype(jnp.int32)
  qcat = query_cat_id.reshape(-1).astype(jnp.int32)
  qbrand = query_brand_id.reshape(-1).astype(jnp.int32)
  mat = match.astype(jnp.int32).reshape(-1)
  id_rows, cat_rows, brand_rows, wa, wb, mrows = _sc_gather(
      qid, qcat, qbrand, mat, id_table, cat_table, brand_table,
      out_table, weight_table.reshape(WROWS, EMB))
  match_rows = mrows.reshape(B, NMATCH * EMB)
  return _tc_compute(id_rows, cat_rows, brand_rows, qid.reshape(B, 1), wa,
                     wb, match_rows)
